# baseline (speedup 1.0000x reference)
import functools

import jax
import jax.numpy as jnp
from jax import lax
from jax.experimental import pallas as pl
from jax.experimental.pallas import tpu as pltpu
from jax.experimental.pallas import tpu_sc as plsc

_BLK = 128


def _sc_lookup(n_fields, batch, vocab, dim, n_workers):
    b_per_w = batch // n_workers
    n_blk = b_per_w // _BLK
    mesh = plsc.VectorSubcoreMesh(core_axis_name="c", subcore_axis_name="s")

    @functools.partial(
        pl.kernel,
        out_type=jax.ShapeDtypeStruct((n_fields, dim, batch), jnp.float32),
        mesh=mesh,
        scratch_types=[
            pltpu.VMEM((n_fields, _BLK), jnp.int32),
            pltpu.VMEM((n_fields, _BLK), jnp.int32),
            pltpu.VMEM((2, _BLK, 2 * dim), jnp.float32),
            pltpu.VMEM((2, dim, _BLK + 1), jnp.float32),
            pltpu.SemaphoreType.DMA,
            pltpu.SemaphoreType.DMA,
            pltpu.SemaphoreType.DMA,
            pltpu.SemaphoreType.DMA,
        ],
        compiler_params=pltpu.CompilerParams(needs_layout_passes=False),
    )
    def k(
        ids_hbm, table_hbm, out_hbm, ids_v, idsh_v, fetch_v, otile_v,
        g0, g1, w0, w1,
    ):
        nc = lax.axis_size("c")
        wid = lax.axis_index("s") * nc + lax.axis_index("c")
        gsem = (g0, g1)
        wsem = (w0, w1)
        iota16 = lax.iota(jnp.int32, 16)
        diag = [(d + iota16) % 16 for d in range(16)]

        def transpose_tile(pb, wb, f):
            src = fetch_v.at[pb]
            dst = otile_v.at[wb]

            @plsc.parallel_loop(0, _BLK, step=16, unroll=2)
            def j_body(jb):
                rows = jb + iota16
                off = (ids_v[f, pl.ds(jb, 16)] & 1) * dim
                for cb in range(0, dim, 16):
                    for d in range(16):
                        cols = cb + diag[d]
                        vals = plsc.load_gather(src, [rows, cols + off])
                        plsc.store_scatter(dst, [cols, rows], vals)

        def fire(f, par, b0):
            return pltpu.async_copy(
                table_hbm.at[idsh_v.at[f]], fetch_v.at[par], gsem[par]
            )

        def drain_gather(par):
            pltpu.make_async_copy(
                table_hbm.at[pl.ds(0, _BLK)], fetch_v.at[par], gsem[par]
            ).wait()

        def drain_write(par, b0):
            pltpu.make_async_copy(
                otile_v.at[par, :, pl.ds(0, _BLK)],
                out_hbm.at[0, :, pl.ds(b0, _BLK)],
                wsem[par],
            ).wait()

        def blk_body(blk, carry):
            b0 = wid * b_per_w + blk * _BLK
            pltpu.sync_copy(ids_hbm.at[:, pl.ds(b0, _BLK)], ids_v)

            @plsc.parallel_loop(0, n_fields * (_BLK // 16), step=1, unroll=4)
            def s_body(u):
                f = u // (_BLK // 16)
                g = u % (_BLK // 16)
                v = ids_v[f, pl.ds(16 * g, 16)]
                idsh_v[f, pl.ds(16 * g, 16)] = lax.shift_right_logical(v, 1)

            fire(0, 0, b0)
            fire(1, 1, b0)

            def f_body(t, carry2):
                for par in range(2):
                    f = 2 * t + par
                    drain_gather(par)

                    @pl.when(t > 0)
                    def _(par=par):
                        drain_write(par, b0)

                    transpose_tile(par, par, f)

                    @pl.when(t < n_fields // 2 - 1)
                    def _(f=f, par=par):
                        fire(f + 2, par, b0)

                    pltpu.async_copy(
                        otile_v.at[par, :, pl.ds(0, _BLK)],
                        out_hbm.at[f, :, pl.ds(b0, _BLK)],
                        wsem[par],
                    )
                return carry2

            lax.fori_loop(0, n_fields // 2, f_body, 0)
            drain_write(0, b0)
            drain_write(1, b0)
            return carry

        lax.fori_loop(0, n_blk, blk_body, 0)

    return k


def _sc_detile(vocab, dim, n_workers):
    vb = 256
    n_blocks = vocab // vb
    tail = vocab - n_blocks * vb
    n_main = (n_blocks // n_workers) * n_workers
    n_mine = n_main // n_workers
    n_rest = n_blocks - n_main
    mesh = plsc.VectorSubcoreMesh(core_axis_name="c", subcore_axis_name="s")

    @functools.partial(
        pl.kernel,
        out_type=jax.ShapeDtypeStruct((vocab // 2, 2 * dim), jnp.float32),
        mesh=mesh,
        scratch_types=[
            pltpu.VMEM((2, dim, vb), jnp.float32),
            pltpu.VMEM((2, vb // 2, 2 * dim), jnp.float32),
            pltpu.SemaphoreType.DMA,
            pltpu.SemaphoreType.DMA,
            pltpu.SemaphoreType.DMA,
            pltpu.SemaphoreType.DMA,
        ],
        compiler_params=pltpu.CompilerParams(needs_layout_passes=False),
    )
    def k(tab_t_hbm, tail_hbm, out_hbm, slab_v, rows_v, r0, r1, w0, w1):
        nc = lax.axis_size("c")
        wid = lax.axis_index("s") * nc + lax.axis_index("c")
        rsem = (r0, r1)
        wsem = (w0, w1)
        iota16 = lax.iota(jnp.int32, 16)
        diag = [(d + iota16) % 16 for d in range(16)]

        def transpose_slab(par):
            src = slab_v.at[par]
            dst = rows_v.at[par]

            @plsc.parallel_loop(0, vb, step=16, unroll=2)
            def v_body(v0):
                rows = v0 + iota16
                prow = lax.shift_right_logical(rows, 1)
                off = (rows & 1) * dim
                for cb in range(0, dim, 16):
                    for d in range(16):
                        cols = cb + diag[d]
                        vals = plsc.load_gather(src, [cols, rows])
                        plsc.store_scatter(dst, [prow, cols + off], vals)

        def rd(b, par):
            return pltpu.async_copy(
                tab_t_hbm.at[:, pl.ds(b * vb, vb)], slab_v.at[par], rsem[par]
            )

        def wr(b, par):
            return pltpu.async_copy(
                rows_v.at[par], out_hbm.at[pl.ds(b * (vb // 2), vb // 2)], wsem[par]
            )

        def drain_rd(par):
            pltpu.make_async_copy(
                tab_t_hbm.at[:, pl.ds(0, vb)], slab_v.at[par], rsem[par]
            ).wait()

        def drain_wr(par):
            pltpu.make_async_copy(
                rows_v.at[par], out_hbm.at[pl.ds(0, vb // 2)], wsem[par]
            ).wait()

        def blk(i):
            return i * n_workers + wid

        rd(blk(0), 0)
        rd(blk(1), 1)

        def t_body(t, carry):
            for par in range(2):
                i = 2 * t + par
                drain_rd(par)

                @pl.when(t > 0)
                def _(par=par):
                    drain_wr(par)

                transpose_slab(par)

                @pl.when(2 * t + par + 2 < n_mine)
                def _(i=i, par=par):
                    rd(blk(i + 2), par)

                wr(blk(i), par)
            return carry

        lax.fori_loop(0, n_mine // 2, t_body, 0)
        drain_wr(0)
        drain_wr(1)

        if n_rest:
            @pl.when(wid < n_rest)
            def _():
                b = n_main + wid
                rd(b, 0).wait()
                transpose_slab(0)
                wr(b, 0).wait()

        if tail:
            @pl.when(wid == n_workers - 1)
            def _():
                p0 = n_blocks * (vb // 2)
                pltpu.sync_copy(
                    tail_hbm,
                    rows_v.at[1, pl.ds(0, tail // 2), pl.ds(0, 2 * dim)],
                )
                pltpu.sync_copy(
                    rows_v.at[1, pl.ds(0, tail // 2), pl.ds(0, 2 * dim)],
                    out_hbm.at[pl.ds(p0, tail // 2)],
                )

    return k


def kernel(ids, table):
    batch, n_fields = ids.shape
    vocab, dim = table.shape
    n_tail = vocab % 256
    tail_pairs = table[vocab - n_tail :].reshape(n_tail // 2, 2 * dim)
    table_rows = _sc_detile(vocab, dim, 32)(table.T, tail_pairs)
    ids_t = ids.T
    out_t = _sc_lookup(n_fields, batch, vocab, dim, 32)(ids_t, table_rows)
    return out_t.transpose(2, 0, 1)


# speedup vs baseline: 1.0608x; 1.0608x over previous
import functools

import jax
import jax.numpy as jnp
from jax import lax
from jax.experimental import pallas as pl
from jax.experimental.pallas import tpu as pltpu
from jax.experimental.pallas import tpu_sc as plsc

_BLK = 128


def _sc_lookup(n_fields, batch, vocab, dim, n_workers):
    b_per_w = batch // n_workers
    n_blk = b_per_w // _BLK
    mesh = plsc.VectorSubcoreMesh(core_axis_name="c", subcore_axis_name="s")

    @functools.partial(
        pl.kernel,
        out_type=jax.ShapeDtypeStruct((n_fields, dim, batch), jnp.float32),
        mesh=mesh,
        scratch_types=[
            pltpu.VMEM((n_fields, _BLK), jnp.int32),
            pltpu.VMEM((2, _BLK, 2 * dim), jnp.float32),
            pltpu.VMEM((2, dim, _BLK + 1), jnp.float32),
            pltpu.SemaphoreType.DMA,
            pltpu.SemaphoreType.DMA,
            pltpu.SemaphoreType.DMA,
            pltpu.SemaphoreType.DMA,
        ],
        compiler_params=pltpu.CompilerParams(needs_layout_passes=False),
    )
    def k(
        ids_hbm, table_hbm, out_hbm, ids_v, fetch_v, otile_v, g0, g1, w0, w1,
    ):
        nc = lax.axis_size("c")
        wid = lax.axis_index("s") * nc + lax.axis_index("c")
        gsem = (g0, g1)
        wsem = (w0, w1)
        iota16 = lax.iota(jnp.int32, 16)
        diag = [(d + iota16) % 16 for d in range(16)]

        def transpose_tile(pb, wb):
            src = fetch_v.at[pb]
            dst = otile_v.at[wb]

            @plsc.parallel_loop(0, _BLK, step=16, unroll=2)
            def j_body(jb):
                rows = jb + iota16
                for cb in range(0, dim, 16):
                    for d in range(16):
                        cols = cb + diag[d]
                        vals = plsc.load_gather(src, [rows, cols])
                        plsc.store_scatter(dst, [cols, rows], vals)

        def fire(f, par, b0):
            return pltpu.async_copy(
                table_hbm.at[ids_v.at[f]], fetch_v.at[par], gsem[par]
            )

        def drain_gather(par):
            pltpu.make_async_copy(
                table_hbm.at[pl.ds(0, _BLK)], fetch_v.at[par], gsem[par]
            ).wait()

        def drain_write(par, b0):
            pltpu.make_async_copy(
                otile_v.at[par, :, pl.ds(0, _BLK)],
                out_hbm.at[0, :, pl.ds(b0, _BLK)],
                wsem[par],
            ).wait()

        def blk_body(blk, carry):
            b0 = wid * b_per_w + blk * _BLK
            pltpu.sync_copy(ids_hbm.at[:, pl.ds(b0, _BLK)], ids_v)
            fire(0, 0, b0)
            fire(1, 1, b0)

            def f_body(t, carry2):
                for par in range(2):
                    f = 2 * t + par
                    drain_gather(par)

                    @pl.when(t > 0)
                    def _(par=par):
                        drain_write(par, b0)

                    transpose_tile(par, par)

                    @pl.when(t < n_fields // 2 - 1)
                    def _(f=f, par=par):
                        fire(f + 2, par, b0)

                    pltpu.async_copy(
                        otile_v.at[par, :, pl.ds(0, _BLK)],
                        out_hbm.at[f, :, pl.ds(b0, _BLK)],
                        wsem[par],
                    )
                return carry2

            lax.fori_loop(0, n_fields // 2, f_body, 0)
            drain_write(0, b0)
            drain_write(1, b0)
            return carry

        lax.fori_loop(0, n_blk, blk_body, 0)

    return k


def _sc_detile(vocab, dim, n_workers):
    vb = 256
    n_blocks = vocab // vb
    tail = vocab - n_blocks * vb
    n_main = (n_blocks // n_workers) * n_workers
    n_mine = n_main // n_workers
    n_rest = n_blocks - n_main
    mesh = plsc.VectorSubcoreMesh(core_axis_name="c", subcore_axis_name="s")

    @functools.partial(
        pl.kernel,
        out_type=jax.ShapeDtypeStruct((vocab, 2 * dim), jnp.float32),
        mesh=mesh,
        scratch_types=[
            pltpu.VMEM((2, dim, vb), jnp.float32),
            pltpu.VMEM((2, vb, 2 * dim), jnp.float32),
            pltpu.SemaphoreType.DMA,
            pltpu.SemaphoreType.DMA,
            pltpu.SemaphoreType.DMA,
            pltpu.SemaphoreType.DMA,
        ],
        compiler_params=pltpu.CompilerParams(needs_layout_passes=False),
    )
    def k(tab_t_hbm, tail_hbm, out_hbm, slab_v, rows_v, r0, r1, w0, w1):
        nc = lax.axis_size("c")
        wid = lax.axis_index("s") * nc + lax.axis_index("c")
        rsem = (r0, r1)
        wsem = (w0, w1)
        iota16 = lax.iota(jnp.int32, 16)
        diag = [(d + iota16) % 16 for d in range(16)]

        def transpose_slab(par):
            src = slab_v.at[par]
            dst = rows_v.at[par]

            @plsc.parallel_loop(0, vb, step=16, unroll=4)
            def v_body(v0):
                rows = v0 + iota16
                for cb in range(0, dim, 16):
                    for d in range(16):
                        cols = cb + diag[d]
                        vals = plsc.load_gather(src, [cols, rows])
                        plsc.store_scatter(dst, [rows, cols], vals)

        def rd(b, par):
            return pltpu.async_copy(
                tab_t_hbm.at[:, pl.ds(b * vb, vb)], slab_v.at[par], rsem[par]
            )

        def wr(b, par):
            return pltpu.async_copy(
                rows_v.at[par], out_hbm.at[pl.ds(b * vb, vb)], wsem[par]
            )

        def drain_rd(par):
            pltpu.make_async_copy(
                tab_t_hbm.at[:, pl.ds(0, vb)], slab_v.at[par], rsem[par]
            ).wait()

        def drain_wr(par):
            pltpu.make_async_copy(
                rows_v.at[par], out_hbm.at[pl.ds(0, vb)], wsem[par]
            ).wait()

        def blk(i):
            return i * n_workers + wid

        rd(blk(0), 0)
        rd(blk(1), 1)

        def t_body(t, carry):
            for par in range(2):
                i = 2 * t + par
                drain_rd(par)

                @pl.when(t > 0)
                def _(par=par):
                    drain_wr(par)

                transpose_slab(par)

                @pl.when(2 * t + par + 2 < n_mine)
                def _(i=i, par=par):
                    rd(blk(i + 2), par)

                wr(blk(i), par)
            return carry

        lax.fori_loop(0, n_mine // 2, t_body, 0)
        drain_wr(0)
        drain_wr(1)

        if n_rest:
            @pl.when(wid < n_rest)
            def _():
                b = n_main + wid
                rd(b, 0).wait()
                transpose_slab(0)
                wr(b, 0).wait()

        if tail:
            @pl.when(wid == n_workers - 1)
            def _():
                v0 = n_blocks * vb
                pltpu.sync_copy(
                    tail_hbm, rows_v.at[1, pl.ds(0, tail), pl.ds(0, 2 * dim)]
                )
                pltpu.sync_copy(
                    rows_v.at[1, pl.ds(0, tail), pl.ds(0, 2 * dim)],
                    out_hbm.at[pl.ds(v0, tail)],
                )

    return k


def kernel(ids, table):
    batch, n_fields = ids.shape
    vocab, dim = table.shape
    n_tail = vocab % 256
    tail_pad = jnp.pad(table[vocab - n_tail :], ((0, 0), (0, dim)))
    table_rows = _sc_detile(vocab, dim, 32)(table.T, tail_pad)
    ids_t = ids.T
    out_t = _sc_lookup(n_fields, batch, vocab, dim, 32)(ids_t, table_rows)
    return out_t.transpose(2, 0, 1)


# speedup vs baseline: 1.2115x; 1.1420x over previous
import functools

import jax
import jax.numpy as jnp
from jax import lax
from jax.experimental import pallas as pl
from jax.experimental.pallas import tpu as pltpu
from jax.experimental.pallas import tpu_sc as plsc

_BLK = 128


def _sc_lookup(n_fields, batch, vocab, dim, n_workers):
    b_per_w = batch // n_workers
    n_blk = b_per_w // _BLK
    mesh = plsc.VectorSubcoreMesh(core_axis_name="c", subcore_axis_name="s")

    @functools.partial(
        pl.kernel,
        out_type=jax.ShapeDtypeStruct((n_fields, dim, batch), jnp.float32),
        mesh=mesh,
        scratch_types=[
            pltpu.VMEM((n_fields, _BLK), jnp.int32),
            pltpu.VMEM((2, _BLK, 2 * dim), jnp.float32),
            pltpu.VMEM((2, dim, _BLK + 1), jnp.float32),
            pltpu.SemaphoreType.DMA,
            pltpu.SemaphoreType.DMA,
            pltpu.SemaphoreType.DMA,
            pltpu.SemaphoreType.DMA,
        ],
        compiler_params=pltpu.CompilerParams(needs_layout_passes=False),
    )
    def k(
        ids_hbm, table_hbm, out_hbm, ids_v, fetch_v, otile_v, g0, g1, w0, w1,
    ):
        nc = lax.axis_size("c")
        wid = lax.axis_index("s") * nc + lax.axis_index("c")
        gsem = (g0, g1)
        wsem = (w0, w1)
        iota16 = lax.iota(jnp.int32, 16)
        diag = [(d + iota16) % 16 for d in range(16)]

        def transpose_tile(pb, wb):
            src = fetch_v.at[pb]
            dst = otile_v.at[wb]

            @plsc.parallel_loop(0, _BLK, step=16, unroll=4)
            def j_body(jb):
                rows = jb + iota16
                for cb in range(0, dim, 16):
                    for d in range(16):
                        cols = cb + diag[d]
                        vals = plsc.load_gather(src, [rows, cols])
                        plsc.store_scatter(dst, [cols, rows], vals)

        def fire(f, par, b0):
            return pltpu.async_copy(
                table_hbm.at[ids_v.at[f]], fetch_v.at[par], gsem[par]
            )

        def drain_gather(par):
            pltpu.make_async_copy(
                table_hbm.at[pl.ds(0, _BLK)], fetch_v.at[par], gsem[par]
            ).wait()

        def drain_write(par, b0):
            pltpu.make_async_copy(
                otile_v.at[par, :, pl.ds(0, _BLK)],
                out_hbm.at[0, :, pl.ds(b0, _BLK)],
                wsem[par],
            ).wait()

        def blk_body(blk, carry):
            b0 = wid * b_per_w + blk * _BLK
            pltpu.sync_copy(ids_hbm.at[:, pl.ds(b0, _BLK)], ids_v)
            fire(0, 0, b0)
            fire(1, 1, b0)

            def f_body(t, carry2):
                for par in range(2):
                    f = 2 * t + par
                    drain_gather(par)

                    @pl.when(t > 0)
                    def _(par=par):
                        drain_write(par, b0)

                    transpose_tile(par, par)

                    @pl.when(t < n_fields // 2 - 1)
                    def _(f=f, par=par):
                        fire(f + 2, par, b0)

                    pltpu.async_copy(
                        otile_v.at[par, :, pl.ds(0, _BLK)],
                        out_hbm.at[f, :, pl.ds(b0, _BLK)],
                        wsem[par],
                    )
                return carry2

            lax.fori_loop(0, n_fields // 2, f_body, 0)
            drain_write(0, b0)
            drain_write(1, b0)
            return carry

        lax.fori_loop(0, n_blk, blk_body, 0)

    return k


def _sc_detile(vocab, dim, n_workers):
    vb = 256
    n_blocks = vocab // vb
    tail = vocab - n_blocks * vb
    n_main = (n_blocks // n_workers) * n_workers
    n_mine = n_main // n_workers
    n_rest = n_blocks - n_main
    mesh = plsc.VectorSubcoreMesh(core_axis_name="c", subcore_axis_name="s")

    @functools.partial(
        pl.kernel,
        out_type=jax.ShapeDtypeStruct((vocab, 2 * dim), jnp.float32),
        mesh=mesh,
        scratch_types=[
            pltpu.VMEM((2, dim, vb), jnp.float32),
            pltpu.VMEM((2, vb, 2 * dim), jnp.float32),
            pltpu.SemaphoreType.DMA,
            pltpu.SemaphoreType.DMA,
            pltpu.SemaphoreType.DMA,
            pltpu.SemaphoreType.DMA,
        ],
        compiler_params=pltpu.CompilerParams(needs_layout_passes=False),
    )
    def k(tab_t_hbm, tail_hbm, out_hbm, slab_v, rows_v, r0, r1, w0, w1):
        nc = lax.axis_size("c")
        wid = lax.axis_index("s") * nc + lax.axis_index("c")
        rsem = (r0, r1)
        wsem = (w0, w1)
        iota16 = lax.iota(jnp.int32, 16)
        diag = [(d + iota16) % 16 for d in range(16)]

        def transpose_slab(par):
            src = slab_v.at[par]
            dst = rows_v.at[par]

            @plsc.parallel_loop(0, vb, step=16, unroll=4)
            def v_body(v0):
                rows = v0 + iota16
                for cb in range(0, dim, 16):
                    for d in range(16):
                        cols = cb + diag[d]
                        vals = plsc.load_gather(src, [cols, rows])
                        plsc.store_scatter(dst, [rows, cols], vals)

        def rd(b, par):
            return pltpu.async_copy(
                tab_t_hbm.at[:, pl.ds(b * vb, vb)], slab_v.at[par], rsem[par]
            )

        def wr(b, par):
            return pltpu.async_copy(
                rows_v.at[par], out_hbm.at[pl.ds(b * vb, vb)], wsem[par]
            )

        def drain_rd(par):
            pltpu.make_async_copy(
                tab_t_hbm.at[:, pl.ds(0, vb)], slab_v.at[par], rsem[par]
            ).wait()

        def drain_wr(par):
            pltpu.make_async_copy(
                rows_v.at[par], out_hbm.at[pl.ds(0, vb)], wsem[par]
            ).wait()

        def blk(i):
            return i * n_workers + wid

        rd(blk(0), 0)
        rd(blk(1), 1)

        def t_body(t, carry):
            for par in range(2):
                i = 2 * t + par
                drain_rd(par)

                @pl.when(t > 0)
                def _(par=par):
                    drain_wr(par)

                transpose_slab(par)

                @pl.when(2 * t + par + 2 < n_mine)
                def _(i=i, par=par):
                    rd(blk(i + 2), par)

                wr(blk(i), par)
            return carry

        lax.fori_loop(0, n_mine // 2, t_body, 0)
        drain_wr(0)
        drain_wr(1)

        if n_rest:
            @pl.when(wid < n_rest)
            def _():
                b = n_main + wid
                rd(b, 0).wait()
                transpose_slab(0)
                wr(b, 0).wait()

        if tail:
            @pl.when(wid == n_workers - 1)
            def _():
                v0 = n_blocks * vb
                pltpu.sync_copy(
                    tail_hbm, rows_v.at[1, pl.ds(0, tail), pl.ds(0, 2 * dim)]
                )
                pltpu.sync_copy(
                    rows_v.at[1, pl.ds(0, tail), pl.ds(0, 2 * dim)],
                    out_hbm.at[pl.ds(v0, tail)],
                )

    return k


def kernel(ids, table):
    batch, n_fields = ids.shape
    vocab, dim = table.shape
    n_tail = vocab % 256
    tail_pad = jnp.pad(table[vocab - n_tail :], ((0, 0), (0, dim)))
    table_rows = _sc_detile(vocab, dim, 32)(table.T, tail_pad)
    ids_t = ids.T
    out_t = _sc_lookup(n_fields, batch, vocab, dim, 32)(ids_t, table_rows)
    return out_t.transpose(2, 0, 1)
